# baseline (speedup 1.0000x reference)
import functools

import jax
import jax.numpy as jnp
from jax import lax
from jax.experimental import pallas as pl
from jax.experimental.pallas import tpu as pltpu
from jax.experimental.pallas import tpu_sc as plsc

L = 16
NB = 4
CH = 4


def _sc_body(B, T, D, x_hbm, pe_hbm, table_hbm, mid_hbm, out_hbm,
             idx_v, me_v, pe_v, x_v, sem_in, sem_out, gsem):
    c = lax.axis_index("c")
    s = lax.axis_index("s")
    nc = lax.axis_size("c")
    ns = lax.axis_size("s")
    nw = nc * ns
    wid = s * nc + c

    pltpu.sync_copy(mid_hbm, idx_v)
    pltpu.async_copy(table_hbm.at[idx_v], me_v, gsem).wait()

    t_per_w = T // nw
    n_ch = t_per_w // CH
    base = wid * t_per_w

    def in_copies(k, sl):
        t0 = base + k * CH
        cps = [pltpu.make_async_copy(
            pe_hbm.at[pl.ds(t0, CH)], pe_v.at[sl], sem_in.at[sl])]
        for b in range(B):
            cps.append(pltpu.make_async_copy(
                x_hbm.at[b, pl.ds(t0, CH), :], x_v.at[sl, b], sem_in.at[sl]))
        return cps

    def out_copies(k, sl):
        t0 = base + k * CH
        return [pltpu.make_async_copy(
            x_v.at[sl, b], out_hbm.at[b, pl.ds(t0, CH), :], sem_out.at[sl])
            for b in range(B)]

    for p in range(NB - 1):
        for cp in in_copies(p, p):
            cp.start()

    def step(k, carry):
        sl = lax.rem(k, NB)
        for cp in in_copies(k, sl):
            cp.wait()

        def jloop(j, carry2):
            slc = pl.ds(j * L, L)
            mv = me_v[0, slc]
            for r in range(CH):
                pv = pe_v[sl, r, slc] + mv
                for b in range(B):
                    x_v[sl, b, r, slc] = x_v[sl, b, r, slc] + pv
            return carry2

        lax.fori_loop(0, D // L, jloop, 0)

        for cp in out_copies(k, sl):
            cp.start()

        nsl = lax.rem(k + NB - 1, NB)

        @pl.when(k + NB - 1 < n_ch)
        def _():
            @pl.when(k >= 1)
            def _():
                for cp in out_copies(k - 1, nsl):
                    cp.wait()
            for cp in in_copies(k + NB - 1, nsl):
                cp.start()

        return carry

    lax.fori_loop(0, n_ch, step, 0)

    for kk in range(n_ch - NB, n_ch):
        for cp in out_copies(kk, kk % NB):
            cp.wait()


@jax.jit
def kernel(x, temporal_pe, modality_table, modality_id):
    B, T, D = x.shape
    pe2 = temporal_pe.reshape(temporal_pe.shape[1], D)
    mid = jnp.asarray(modality_id, jnp.int32).reshape(1)

    mesh = plsc.VectorSubcoreMesh(core_axis_name="c", subcore_axis_name="s")
    body = functools.partial(_sc_body, B, T, D)
    k = pl.kernel(
        body,
        mesh=mesh,
        out_type=jax.ShapeDtypeStruct((B, T, D), x.dtype),
        scratch_types=[
            pltpu.VMEM((1,), jnp.int32),
            pltpu.VMEM((1, D), jnp.float32),
            pltpu.VMEM((NB, CH, D), jnp.float32),
            pltpu.VMEM((NB, B, CH, D), jnp.float32),
            pltpu.SemaphoreType.DMA((NB,)),
            pltpu.SemaphoreType.DMA((NB,)),
            pltpu.SemaphoreType.DMA,
        ],
    )
    return k(x, pe2, modality_table, mid)


# speedup vs baseline: 2.2660x; 2.2660x over previous
import functools

import jax
import jax.numpy as jnp
from jax import lax
from jax.experimental import pallas as pl
from jax.experimental.pallas import tpu as pltpu
from jax.experimental.pallas import tpu_sc as plsc

L = 16
NB = 4
CH = 4


def _sc_body(B, T, D, x_hbm, pe_hbm, table_hbm, mid_hbm, out_hbm,
             idx_v, me_v, pe_v, x_v, sem_in, sem_out, gsem):
    c = lax.axis_index("c")
    s = lax.axis_index("s")
    nc = lax.axis_size("c")
    ns = lax.axis_size("s")
    nw = nc * ns
    wid = s * nc + c

    pltpu.sync_copy(mid_hbm, idx_v)
    pltpu.async_copy(table_hbm.at[idx_v], me_v, gsem).wait()

    t_per_w = T // nw
    n_ch = t_per_w // CH
    base = wid * t_per_w

    def in_copies(k, sl):
        t0 = base + k * CH
        cps = [pltpu.make_async_copy(
            pe_hbm.at[pl.ds(t0, CH)], pe_v.at[sl], sem_in.at[sl])]
        for b in range(B):
            cps.append(pltpu.make_async_copy(
                x_hbm.at[b, pl.ds(t0, CH), :], x_v.at[sl, b], sem_in.at[sl]))
        return cps

    def out_copies(k, sl):
        t0 = base + k * CH
        return [pltpu.make_async_copy(
            x_v.at[sl, b], out_hbm.at[b, pl.ds(t0, CH), :], sem_out.at[sl])
            for b in range(B)]

    def compute(sl):
        pe_sl = pe_v.at[sl]
        x_sl = x_v.at[sl]

        @plsc.parallel_loop(0, D // L, unroll=4)
        def _(j):
            slc = pl.ds(j * L, L)
            mv = me_v[0, slc]
            for r in range(CH):
                pv = pe_sl[r, slc] + mv
                for b in range(B):
                    x_sl[b, r, slc] = x_sl[b, r, slc] + pv

    def step(k, sl, wait_prev_out, issue_next):
        for cp in in_copies(k, sl):
            cp.wait()
        compute(sl)
        for cp in out_copies(k, sl):
            cp.start()
        nsl = (sl + NB - 1) % NB
        if wait_prev_out:
            for cp in out_copies(k - 1, nsl):
                cp.wait()
        if issue_next:
            for cp in in_copies(k + NB - 1, nsl):
                cp.start()

    for p in range(NB - 1):
        for cp in in_copies(p, p):
            cp.start()

    step(0, 0, False, True)
    for k0 in range(1, NB):
        step(k0, k0, True, True)

    def steady(g, carry):
        for sl in range(NB):
            step(g * NB + sl, sl, True, True)
        return carry

    lax.fori_loop(1, n_ch // NB - 1, steady, 0)

    kL = n_ch - NB
    step(kL, 0, True, True)
    for sl in range(1, NB):
        step(kL + sl, sl, False, False)

    for kk in range(n_ch - NB, n_ch):
        for cp in out_copies(kk, kk % NB):
            cp.wait()


@jax.jit
def kernel(x, temporal_pe, modality_table, modality_id):
    B, T, D = x.shape
    pe2 = temporal_pe.reshape(temporal_pe.shape[1], D)
    mid = jnp.asarray(modality_id, jnp.int32).reshape(1)

    mesh = plsc.VectorSubcoreMesh(core_axis_name="c", subcore_axis_name="s")
    body = functools.partial(_sc_body, B, T, D)
    k = pl.kernel(
        body,
        mesh=mesh,
        out_type=jax.ShapeDtypeStruct((B, T, D), x.dtype),
        scratch_types=[
            pltpu.VMEM((1,), jnp.int32),
            pltpu.VMEM((1, D), jnp.float32),
            pltpu.VMEM((NB, CH, D), jnp.float32),
            pltpu.VMEM((NB, B, CH, D), jnp.float32),
            pltpu.SemaphoreType.DMA((NB,)),
            pltpu.SemaphoreType.DMA((NB,)),
            pltpu.SemaphoreType.DMA,
        ],
    )
    return k(x, pe2, modality_table, mid)


# speedup vs baseline: 2.2723x; 1.0028x over previous
import functools

import jax
import jax.numpy as jnp
from jax import lax
from jax.experimental import pallas as pl
from jax.experimental.pallas import tpu as pltpu
from jax.experimental.pallas import tpu_sc as plsc

L = 16
NB = 4
CH = 4


def _sc_body(B, T, D, x_hbm, pe_hbm, table_hbm, mid_hbm, out_hbm,
             idx_v, me_v, pe_v, x_v, sem_in, sem_out, gsem):
    c = lax.axis_index("c")
    s = lax.axis_index("s")
    nc = lax.axis_size("c")
    ns = lax.axis_size("s")
    nw = nc * ns
    wid = s * nc + c

    pltpu.sync_copy(mid_hbm, idx_v)
    pltpu.async_copy(table_hbm.at[idx_v], me_v, gsem).wait()

    t_per_w = T // nw
    n_ch = t_per_w // CH
    base = wid * t_per_w

    def in_copies(k, sl):
        t0 = base + k * CH
        return [
            pltpu.make_async_copy(
                pe_hbm.at[pl.ds(t0, CH)], pe_v.at[sl], sem_in.at[sl]),
            pltpu.make_async_copy(
                x_hbm.at[:, pl.ds(t0, CH), :], x_v.at[sl], sem_in.at[sl]),
        ]

    def out_copies(k, sl):
        t0 = base + k * CH
        return [pltpu.make_async_copy(
            x_v.at[sl, b], out_hbm.at[b, pl.ds(t0, CH), :], sem_out.at[sl])
            for b in range(B)]

    def compute(sl):
        pe_sl = pe_v.at[sl]
        x_sl = x_v.at[sl]

        @plsc.parallel_loop(0, D // L, unroll=8)
        def _(j):
            slc = pl.ds(j * L, L)
            mv = me_v[0, slc]
            for r in range(CH):
                pv = pe_sl[r, slc] + mv
                for b in range(B):
                    plsc.addupdate(x_sl.at[b, r, slc], pv)

    def step(k, sl, wait_prev_out, issue_next):
        for cp in in_copies(k, sl):
            cp.wait()
        compute(sl)
        for cp in out_copies(k, sl):
            cp.start()
        nsl = (sl + NB - 1) % NB
        if wait_prev_out:
            for cp in out_copies(k - 1, nsl):
                cp.wait()
        if issue_next:
            for cp in in_copies(k + NB - 1, nsl):
                cp.start()

    for p in range(NB - 1):
        for cp in in_copies(p, p):
            cp.start()

    step(0, 0, False, True)
    for k0 in range(1, NB):
        step(k0, k0, True, True)

    def steady(g, carry):
        for sl in range(NB):
            step(g * NB + sl, sl, True, True)
        return carry

    lax.fori_loop(1, n_ch // NB - 1, steady, 0)

    kL = n_ch - NB
    step(kL, 0, True, True)
    for sl in range(1, NB):
        step(kL + sl, sl, False, False)

    for kk in range(n_ch - NB, n_ch):
        for cp in out_copies(kk, kk % NB):
            cp.wait()


@jax.jit
def kernel(x, temporal_pe, modality_table, modality_id):
    B, T, D = x.shape
    pe2 = temporal_pe.reshape(temporal_pe.shape[1], D)
    mid = jnp.asarray(modality_id, jnp.int32).reshape(1)

    mesh = plsc.VectorSubcoreMesh(core_axis_name="c", subcore_axis_name="s")
    body = functools.partial(_sc_body, B, T, D)
    k = pl.kernel(
        body,
        mesh=mesh,
        out_type=jax.ShapeDtypeStruct((B, T, D), x.dtype),
        scratch_types=[
            pltpu.VMEM((1,), jnp.int32),
            pltpu.VMEM((1, D), jnp.float32),
            pltpu.VMEM((NB, CH, D), jnp.float32),
            pltpu.VMEM((NB, B, CH, D), jnp.float32),
            pltpu.SemaphoreType.DMA((NB,)),
            pltpu.SemaphoreType.DMA((NB,)),
            pltpu.SemaphoreType.DMA,
        ],
    )
    return k(x, pe2, modality_table, mid)
